# baseline (speedup 1.0000x reference)
import functools
import math

import jax
import jax.numpy as jnp
from jax import lax
from jax.experimental import pallas as pl
from jax.experimental.pallas import tpu as pltpu
from jax.experimental.pallas import tpu_sc as plsc

_THETA = math.log(1.4) - 3.0

_NC = 2
_NS = 16
_NW = _NC * _NS
_SUB = 80
_NSUB = 5
_C = _SUB * _NSUB
_NPAD = 10240
_SL = _NPAD // _NS


def _p_body(x_ref, w_ref, gw_ref, o_ref):
    p = jnp.dot(x_ref[...], w_ref[...], preferred_element_type=jnp.float32)
    sp1 = jnp.sum(p, axis=1, keepdims=True)
    sp2 = jnp.sum(p * p, axis=1, keepdims=True)
    spw = jnp.dot(p, gw_ref[...], preferred_element_type=jnp.float32)
    pad = jnp.zeros((p.shape[0], 13), jnp.float32)
    o_ref[...] = jnp.concatenate([p, sp1, sp2, spw, pad], axis=1)


def _a_body(ef_ref, ts_ref, lu_ref, tw_ref, tb_ref, w_ref, b0_ref, gw_ref,
            olo_ref, ohi_ref, s1_ref, s2_ref, sw_ref):
    dt = ts_ref[...] - lu_ref[...]
    t_code = jnp.cos(dt * tw_ref[...] + tb_ref[...])
    m = ef_ref[...] + t_code
    a = (jnp.dot(m, w_ref[...], preferred_element_type=jnp.float32)
         + b0_ref[...])
    mh = a.shape[1] // 2
    olo_ref[...] = a[:, :mh]
    ohi_ref[...] = a[:, mh:]
    s1_ref[...] = jnp.sum(a, axis=1, keepdims=True)
    s2_ref[...] = jnp.sum(a * a, axis=1, keepdims=True)
    sw_ref[...] = jnp.dot(a, gw_ref[...], preferred_element_type=jnp.float32)


def _fin_body(alo_ref, ahi_ref, deg_ref, p_ref, mem_ref, cnt_ref,
              o_ref, rem_ref):
    acc = jnp.concatenate([alo_ref[0] + alo_ref[1],
                           ahi_ref[0] + ahi_ref[1]], axis=1)
    dall = deg_ref[...]
    d = dall[:, 0:1] + dall[:, 1:2]
    num = acc / jnp.maximum(d, 1.0) + p_ref[...][:, :acc.shape[1]]
    o_ref[...] = jnp.where(d > 0.0, num, mem_ref[...])
    rem_ref[0, 0] = jnp.sum(cnt_ref[...])


def _zero_vmem(ref, n_vecs):
    z = jnp.zeros((16,), jnp.float32)
    nrow = ref.shape[0]
    per_row = (ref.shape[1] // 16) if len(ref.shape) == 2 else 1

    def body(t, _):
        if len(ref.shape) == 2:
            r = t // per_row
            c = (t % per_row) * 16
            ref[r, pl.ds(c, 16)] = z
        else:
            ref[pl.ds(t * 16, 16)] = z
        return 0

    lax.fori_loop(0, n_vecs, body, 0, unroll=4)


def _sc_lu_gather(E):
    ew = E // _NW
    mesh = plsc.VectorSubcoreMesh(core_axis_name="c", subcore_axis_name="s")

    @functools.partial(
        pl.kernel, mesh=mesh,
        compiler_params=pltpu.CompilerParams(needs_layout_passes=False,
                                             use_tc_tiling_on_sc=False),
        out_type=jax.ShapeDtypeStruct((E,), jnp.float32),
        scratch_types=[
            pltpu.VMEM((_NSUB, _SUB), jnp.int32),
            pltpu.VMEM((_C,), jnp.float32),
            pltpu.SemaphoreType.DMA,
        ])
    def gather(dst_hbm, lu_hbm, out_hbm, idx2, val_v, sem):
        wid = lax.axis_index("s") * _NC + lax.axis_index("c")

        def chunk(k, _):
            base = wid * ew + k * _C
            for j in range(_NSUB):
                pltpu.sync_copy(dst_hbm.at[pl.ds(base + j * _SUB, _SUB)],
                                idx2.at[j])
            cps = [pltpu.async_copy(lu_hbm.at[idx2.at[j]],
                                    val_v.at[pl.ds(j * _SUB, _SUB)], sem)
                   for j in range(_NSUB)]
            for cp in cps:
                cp.wait()
            pltpu.sync_copy(val_v, out_hbm.at[pl.ds(base, _C)])
            return 0

        lax.fori_loop(0, ew // _C, chunk, 0)

    return gather


def _sc_main(E, N, MD):
    ew = E // _NW
    MH = MD // 2
    mesh = plsc.VectorSubcoreMesh(core_axis_name="c", subcore_axis_name="s")

    @functools.partial(
        pl.kernel, mesh=mesh,
        compiler_params=pltpu.CompilerParams(needs_layout_passes=False,
                                             use_tc_tiling_on_sc=False),
        out_type=(
            jax.ShapeDtypeStruct((E,), jnp.float32),
            jax.ShapeDtypeStruct((_NW, 16), jnp.float32),
        ),
        scratch_types=[
            pltpu.VMEM((_NSUB, _SUB), jnp.int32),
            pltpu.VMEM((_C, MH), jnp.float32),
            pltpu.VMEM((_C, MH), jnp.float32),
            pltpu.VMEM((_C, MD + 16), jnp.float32),
            pltpu.VMEM((_C,), jnp.float32),
            pltpu.VMEM((_C,), jnp.float32),
            pltpu.VMEM((_C,), jnp.float32),
            pltpu.VMEM((_C,), jnp.float32),
            pltpu.VMEM((16,), jnp.float32),
            pltpu.VMEM((16,), jnp.float32),
            pltpu.SemaphoreType.DMA,
        ])
    def main(alo_hbm, ahi_hbm, p_hbm, dst_hbm, sa1_hbm, sa2_hbm, saw_hbm,
             par_hbm, s_hbm, cnt_hbm,
             idx2, alo_v, ahi_v, p_v, sa1_v, sa2_v, saw_v, s_v, racc,
             par_v, sem):
        cid = lax.axis_index("c")
        sid = lax.axis_index("s")
        wid = sid * _NC + cid

        racc[...] = jnp.zeros((16,), jnp.float32)
        pltpu.sync_copy(par_hbm, par_v)

        rows0 = lax.iota(jnp.int32, 16)
        pvec = par_v[pl.ds(0, 16)]
        wg = pvec[0]
        kc = pvec[1]
        k2 = pvec[2]
        fmd = float(MD)

        def chunk0(k, _):
            base = wid * ew + k * _C
            for j in range(_NSUB):
                pltpu.sync_copy(dst_hbm.at[pl.ds(base + j * _SUB, _SUB)],
                                idx2.at[j])
            pltpu.sync_copy(alo_hbm.at[pl.ds(base, _C)], alo_v)
            pltpu.sync_copy(ahi_hbm.at[pl.ds(base, _C)], ahi_v)
            pltpu.sync_copy(sa1_hbm.at[pl.ds(base, _C)], sa1_v)
            pltpu.sync_copy(sa2_hbm.at[pl.ds(base, _C)], sa2_v)
            pltpu.sync_copy(saw_hbm.at[pl.ds(base, _C)], saw_v)
            cps = [pltpu.async_copy(p_hbm.at[idx2.at[j]],
                                    p_v.at[pl.ds(j * _SUB, _SUB)], sem)
                   for j in range(_NSUB)]
            for cp in cps:
                cp.wait()

            def group(g, _):
                rows = rows0 + g * 16
                zz = jnp.zeros((16,), jnp.float32)

                def make_dot(a_ref, off):
                    def dot(j2, x):
                        ca = jnp.full((16,), j2, jnp.int32)
                        cpi = jnp.full((16,), off + j2, jnp.int32)
                        av = plsc.load_gather(a_ref, [rows, ca])
                        pv = plsc.load_gather(p_v, [rows, cpi])
                        return x + av * pv
                    return dot

                x = lax.fori_loop(0, MH, make_dot(alo_v, 0), zz, unroll=8)
                x = lax.fori_loop(0, MH, make_dot(ahi_v, MH), x, unroll=8)
                n1 = plsc.load_gather(p_v, [rows, jnp.full((16,), MD,
                                                           jnp.int32)])
                n2 = plsc.load_gather(p_v, [rows, jnp.full((16,), MD + 1,
                                                           jnp.int32)])
                n3 = plsc.load_gather(p_v, [rows, jnp.full((16,), MD + 2,
                                                           jnp.int32)])
                s1 = sa1_v[pl.ds(g * 16, 16)] + n1
                s2 = sa2_v[pl.ds(g * 16, 16)] + 2.0 * x + n2
                sw = saw_v[pl.ds(g * 16, 16)] + n3
                v2 = fmd * s2 - s1 * s1 + (fmd * fmd * 1e-5)
                d2 = fmd * sw - s1 * wg
                lhs = d2 * d2
                rhs = jnp.full((16,), k2) * v2
                s_neg = jnp.logical_or(d2 >= 0.0, lhs < rhs)
                s_pos = jnp.logical_and(d2 > 0.0, lhs > rhs)
                kv = jnp.full((16,), kc)
                sf = jnp.where(kv <= 0.0, s_neg, s_pos).astype(jnp.float32)
                s_v[pl.ds(g * 16, 16)] = sf
                racc[...] = racc[...] + sf
                return 0

            lax.fori_loop(0, _C // 16, group, 0)
            pltpu.sync_copy(s_v, s_hbm.at[pl.ds(base, _C)])
            return 0

        lax.fori_loop(0, ew // _C, chunk0, 0)
        pltpu.sync_copy(racc, cnt_hbm.at[wid])

    return main


def _sc_scatter(E, N, MD, with_deg):
    ew = E // _NW
    MH = MD // 2
    mesh = plsc.VectorSubcoreMesh(core_axis_name="c", subcore_axis_name="s")

    acc_t = jax.ShapeDtypeStruct((_NC, _NPAD, MH), jnp.float32)
    deg_t = jax.ShapeDtypeStruct((_NC, _NPAD), jnp.float32)
    scratch = [
        pltpu.VMEM((_NSUB, _SUB), jnp.int32),
        pltpu.VMEM((_C, MH), jnp.float32),
        pltpu.VMEM((_SUB, MH), jnp.float32),
        pltpu.VMEM_SHARED((_NPAD, MH), jnp.float32),
    ]
    if with_deg:
        scratch += [
            pltpu.VMEM((_SUB,), jnp.float32),
            pltpu.VMEM((_SL,), jnp.float32),
            pltpu.VMEM_SHARED((_NPAD,), jnp.float32),
        ]

    @functools.partial(
        pl.kernel, mesh=mesh,
        compiler_params=pltpu.CompilerParams(needs_layout_passes=False,
                                             use_tc_tiling_on_sc=False),
        out_type=(acc_t, deg_t) if with_deg else acc_t,
        scratch_types=scratch)
    def scat(a_hbm, dst_hbm, *refs):
        if with_deg:
            (acc_hbm, deg_hbm, idx2, a_v, zacc, acc_sh,
             ones1, zdeg, deg_sh) = refs
        else:
            acc_hbm, idx2, a_v, zacc, acc_sh = refs
        cid = lax.axis_index("c")
        sid = lax.axis_index("s")
        wid = sid * _NC + cid

        _zero_vmem(zacc, _SUB * MH // 16)
        for z in range(_SL // _SUB):
            pltpu.sync_copy(zacc,
                            acc_sh.at[pl.ds(sid * _SL + z * _SUB, _SUB)])
        if with_deg:
            _zero_vmem(zdeg, _SL // 16)

            def fill_ones(t, _):
                ones1[pl.ds(t * 16, 16)] = jnp.full((16,), 1.0, jnp.float32)
                return 0

            lax.fori_loop(0, _SUB // 16, fill_ones, 0)
            pltpu.sync_copy(zdeg, deg_sh.at[pl.ds(sid * _SL, _SL)])
        plsc.subcore_barrier()

        def chunk1(k, _):
            base = wid * ew + k * _C
            for j in range(_NSUB):
                pltpu.sync_copy(dst_hbm.at[pl.ds(base + j * _SUB, _SUB)],
                                idx2.at[j])
            pltpu.sync_copy(a_hbm.at[pl.ds(base, _C)], a_v)
            for j in range(_NSUB):
                pltpu.sync_copy(a_v.at[pl.ds(j * _SUB, _SUB)],
                                acc_sh.at[idx2.at[j]], add=True)
                if with_deg:
                    pltpu.sync_copy(ones1, deg_sh.at[idx2.at[j]], add=True)
            return 0

        lax.fori_loop(0, ew // _C, chunk1, 0)
        plsc.subcore_barrier()
        pltpu.sync_copy(acc_sh.at[pl.ds(sid * _SL, _SL)],
                        acc_hbm.at[cid, pl.ds(sid * _SL, _SL)])
        if with_deg:
            pltpu.sync_copy(deg_sh.at[pl.ds(sid * _SL, _SL)],
                            deg_hbm.at[cid, pl.ds(sid * _SL, _SL)])

    return scat


def kernel(edge_feat, timestamp, dst_idx, memory, last_update,
           W0, b0, ln_g, ln_b, W1, b1, time_w, time_b):
    E, EF = edge_feat.shape
    N, MD = memory.shape
    f32 = jnp.float32

    dst_i32 = dst_idx.astype(jnp.int32)
    w = (ln_g * W1[0]).astype(f32)
    gw = w.reshape(MD, 1)

    w0b_t = W0[:, EF:].T
    P = pl.pallas_call(
        _p_body,
        out_shape=jax.ShapeDtypeStruct((N, MD + 16), f32),
    )(memory, w0b_t, gw)

    lu_g = _sc_lu_gather(E)(dst_i32, last_update)

    BE = 1280
    w0a_t = W0[:, :EF].T
    grid = (E // BE,)
    A = pl.pallas_call(
        _a_body,
        grid=grid,
        in_specs=[
            pl.BlockSpec((BE, EF), lambda i: (i, 0)),
            pl.BlockSpec((BE, 1), lambda i: (i, 0)),
            pl.BlockSpec((BE, 1), lambda i: (i, 0)),
            pl.BlockSpec((1, EF), lambda i: (0, 0)),
            pl.BlockSpec((1, EF), lambda i: (0, 0)),
            pl.BlockSpec((EF, MD), lambda i: (0, 0)),
            pl.BlockSpec((1, MD), lambda i: (0, 0)),
            pl.BlockSpec((MD, 1), lambda i: (0, 0)),
        ],
        out_specs=(pl.BlockSpec((BE, MD // 2), lambda i: (i, 0)),
                   pl.BlockSpec((BE, MD // 2), lambda i: (i, 0)),
                   pl.BlockSpec((BE, 1), lambda i: (i, 0)),
                   pl.BlockSpec((BE, 1), lambda i: (i, 0)),
                   pl.BlockSpec((BE, 1), lambda i: (i, 0))),
        out_shape=(jax.ShapeDtypeStruct((E, MD // 2), f32),
                   jax.ShapeDtypeStruct((E, MD // 2), f32),
                   jax.ShapeDtypeStruct((E, 1), f32),
                   jax.ShapeDtypeStruct((E, 1), f32),
                   jax.ShapeDtypeStruct((E, 1), f32)),
    )(edge_feat, timestamp.reshape(E, 1), lu_g.reshape(E, 1),
      time_w.reshape(1, EF), time_b.reshape(1, EF), w0a_t,
      b0.reshape(1, MD), gw)
    A_lo, A_hi, sa1, sa2, saw = A

    wg = jnp.sum(w)
    c0 = jnp.sum(ln_b * W1[0]) + b1[0]
    kcv = jnp.float32(_THETA) - c0
    par = jnp.concatenate([jnp.stack([wg, kcv, kcv * kcv]),
                           jnp.zeros((13,))]).astype(f32)
    s_ij, counts = _sc_main(E, N, MD)(
        A_lo, A_hi, P, dst_i32, sa1.reshape(E), sa2.reshape(E),
        saw.reshape(E), par)
    acc_lo, deg = _sc_scatter(E, N, MD, True)(A_lo, dst_i32)
    acc_hi = _sc_scatter(E, N, MD, False)(A_hi, dst_i32)

    BR = 2000
    new_memory, rem = pl.pallas_call(
        _fin_body,
        grid=(N // BR,),
        in_specs=[
            pl.BlockSpec((_NC, BR, MD // 2), lambda i: (0, i, 0)),
            pl.BlockSpec((_NC, BR, MD // 2), lambda i: (0, i, 0)),
            pl.BlockSpec((BR, _NC), lambda i: (i, 0)),
            pl.BlockSpec((BR, MD + 16), lambda i: (i, 0)),
            pl.BlockSpec((BR, MD), lambda i: (i, 0)),
            pl.BlockSpec((_NW, 16), lambda i: (0, 0)),
        ],
        out_shape=(jax.ShapeDtypeStruct((N, MD), f32),
                   jax.ShapeDtypeStruct((1, 1), f32)),
        out_specs=(pl.BlockSpec((BR, MD), lambda i: (i, 0)),
                   pl.BlockSpec(memory_space=pltpu.SMEM)),
    )(acc_lo, acc_hi, deg.T, P, memory, counts)

    penalty = jnp.float32(0.0)
    remain_edge_batch = rem[0, 0]
    total_edge_batch = E
    return (new_memory, penalty, remain_edge_batch, total_edge_batch, s_ij)


# speedup vs baseline: 1.1353x; 1.1353x over previous
import functools
import math

import jax
import jax.numpy as jnp
from jax import lax
from jax.experimental import pallas as pl
from jax.experimental.pallas import tpu as pltpu
from jax.experimental.pallas import tpu_sc as plsc

_THETA = math.log(1.4) - 3.0

_NC = 2
_NS = 16
_NW = _NC * _NS
_SUB = 80
_NSUB = 5
_C = _SUB * _NSUB
_NPAD = 10240
_SL = _NPAD // _NS


def _p_body(x_ref, w_ref, o_ref):
    o_ref[...] = jnp.dot(x_ref[...], w_ref[...],
                         preferred_element_type=jnp.float32)


def _a_body(ef_ref, ts_ref, lu_ref, tw_ref, tb_ref, w_ref, b0_ref,
            olo_ref, ohi_ref):
    dt = ts_ref[...] - lu_ref[...]
    t_code = jnp.cos(dt * tw_ref[...] + tb_ref[...])
    m = ef_ref[...] + t_code
    a = (jnp.dot(m, w_ref[...], preferred_element_type=jnp.float32)
         + b0_ref[...])
    mh = a.shape[1] // 2
    olo_ref[...] = a[:, :mh]
    ohi_ref[...] = a[:, mh:]


def _fin_body(alo_ref, ahi_ref, deg_ref, p_ref, mem_ref, cnt_ref,
              o_ref, rem_ref):
    acc = jnp.concatenate([alo_ref[0] + alo_ref[1],
                           ahi_ref[0] + ahi_ref[1]], axis=1)
    dall = deg_ref[...]
    d = dall[:, 0:1] + dall[:, 1:2]
    num = acc / jnp.maximum(d, 1.0) + p_ref[...][:, :acc.shape[1]]
    o_ref[...] = jnp.where(d > 0.0, num, mem_ref[...])
    rem_ref[0, 0] = jnp.sum(cnt_ref[...])


def _zero_vmem(ref, n_vecs):
    z = jnp.zeros((16,), jnp.float32)
    nrow = ref.shape[0]
    per_row = (ref.shape[1] // 16) if len(ref.shape) == 2 else 1

    def body(t, _):
        if len(ref.shape) == 2:
            r = t // per_row
            c = (t % per_row) * 16
            ref[r, pl.ds(c, 16)] = z
        else:
            ref[pl.ds(t * 16, 16)] = z
        return 0

    lax.fori_loop(0, n_vecs, body, 0, unroll=4)


def _sc_lu_gather(E):
    ew = E // _NW
    NR = ew // _SUB
    mesh = plsc.VectorSubcoreMesh(core_axis_name="c", subcore_axis_name="s")

    @functools.partial(
        pl.kernel, mesh=mesh,
        compiler_params=pltpu.CompilerParams(needs_layout_passes=False,
                                             use_tc_tiling_on_sc=False),
        out_type=jax.ShapeDtypeStruct((E,), jnp.float32),
        scratch_types=[
            pltpu.VMEM((NR, _SUB), jnp.int32),
            pltpu.VMEM((_C,), jnp.float32),
            pltpu.SemaphoreType.DMA,
        ])
    def gather(dst2_hbm, lu_hbm, out_hbm, idx_all, val_v, sem):
        wid = lax.axis_index("s") * _NC + lax.axis_index("c")
        pltpu.sync_copy(dst2_hbm.at[pl.ds(wid * NR, NR)], idx_all)

        def chunk(k, _):
            base = wid * ew + k * _C
            cps = [pltpu.async_copy(lu_hbm.at[idx_all.at[k * _NSUB + j]],
                                    val_v.at[pl.ds(j * _SUB, _SUB)], sem)
                   for j in range(_NSUB)]
            for cp in cps:
                cp.wait()
            pltpu.sync_copy(val_v, out_hbm.at[pl.ds(base, _C)])
            return 0

        lax.fori_loop(0, ew // _C, chunk, 0)

    return gather


def _sc_main(E, N, MD):
    ew = E // _NW
    MH = MD // 2
    C = _SUB
    KCH = ew // C
    NR = ew // _SUB
    mesh = plsc.VectorSubcoreMesh(core_axis_name="c", subcore_axis_name="s")

    @functools.partial(
        pl.kernel, mesh=mesh,
        compiler_params=pltpu.CompilerParams(needs_layout_passes=False,
                                             use_tc_tiling_on_sc=False),
        out_type=(
            jax.ShapeDtypeStruct((E,), jnp.float32),
            jax.ShapeDtypeStruct((_NW, 16), jnp.float32),
        ),
        scratch_types=[
            pltpu.VMEM((NR, _SUB), jnp.int32),
            pltpu.VMEM((2, C, MH), jnp.float32),
            pltpu.VMEM((2, C, MH), jnp.float32),
            pltpu.VMEM((2, C, MD), jnp.float32),
            pltpu.VMEM((2, C), jnp.float32),
            pltpu.VMEM((16,), jnp.float32),
            pltpu.VMEM((MD,), jnp.float32),
            pltpu.VMEM((16,), jnp.float32),
            pltpu.SemaphoreType.DMA,
            pltpu.SemaphoreType.DMA,
            pltpu.SemaphoreType.DMA,
            pltpu.SemaphoreType.DMA,
        ])
    def main(alo_hbm, ahi_hbm, p_hbm, dst2_hbm, w_hbm, par_hbm,
             s_hbm, cnt_hbm,
             idx_all, alo_v, ahi_v, p_v, s_v, racc, w_v, par_v,
             si0, si1, so0, so1):
        cid = lax.axis_index("c")
        sid = lax.axis_index("s")
        wid = sid * _NC + cid
        sem_in = (si0, si1)
        sem_out = (so0, so1)

        racc[...] = jnp.zeros((16,), jnp.float32)
        pltpu.sync_copy(par_hbm, par_v)
        pltpu.sync_copy(w_hbm, w_v)
        pltpu.sync_copy(dst2_hbm.at[pl.ds(wid * NR, NR)], idx_all)

        rows0 = lax.iota(jnp.int32, 16)
        pvec = par_v[pl.ds(0, 16)]
        wg = pvec[0]
        kc = pvec[1]
        k2 = pvec[2]
        fmd = float(MD)

        def prefetch(k, b):
            base = wid * ew + k * C
            pltpu.async_copy(alo_hbm.at[pl.ds(base, C)], alo_v.at[b],
                             sem_in[b])
            pltpu.async_copy(ahi_hbm.at[pl.ds(base, C)], ahi_v.at[b],
                             sem_in[b])
            pltpu.async_copy(p_hbm.at[idx_all.at[k]], p_v.at[b], sem_in[b])

        def drain_in(b):
            pltpu.make_async_copy(alo_hbm.at[pl.ds(0, C)], alo_v.at[b],
                                  sem_in[b]).wait()
            pltpu.make_async_copy(ahi_hbm.at[pl.ds(0, C)], ahi_v.at[b],
                                  sem_in[b]).wait()
            pltpu.make_async_copy(p_hbm.at[pl.ds(0, C)], p_v.at[b],
                                  sem_in[b]).wait()

        def compute(k, b):
            @pl.when(k >= 2)
            def _():
                pltpu.make_async_copy(s_v.at[b], s_hbm.at[pl.ds(0, C)],
                                      sem_out[b]).wait()

            def group(g, _):
                rows = rows0 + g * 16
                zz = jnp.zeros((16,), jnp.float32)

                def make_feat(a_ref, off):
                    def feat(jv, carry):
                        s1, s2, sw = carry
                        wvec = w_v[pl.ds(off + jv * 16, 16)]
                        for t in range(16):
                            ca = jnp.full((16,), jv * 16 + t, jnp.int32)
                            cpi = jnp.full((16,), off + jv * 16 + t,
                                           jnp.int32)
                            av = plsc.load_gather(a_ref, [rows, ca])
                            pv = plsc.load_gather(p_v.at[b], [rows, cpi])
                            h = av + pv
                            s1 = s1 + h
                            s2 = s2 + h * h
                            sw = sw + h * wvec[t]
                        return (s1, s2, sw)
                    return feat

                car = lax.fori_loop(0, MH // 16, make_feat(alo_v.at[b], 0),
                                    (zz, zz, zz))
                s1, s2, sw = lax.fori_loop(
                    0, MH // 16, make_feat(ahi_v.at[b], MH), car)
                v2 = fmd * s2 - s1 * s1 + (fmd * fmd * 1e-5)
                d2 = fmd * sw - s1 * wg
                lhs = d2 * d2
                rhs = jnp.full((16,), k2) * v2
                s_neg = jnp.logical_or(d2 >= 0.0, lhs < rhs)
                s_pos = jnp.logical_and(d2 > 0.0, lhs > rhs)
                kv = jnp.full((16,), kc)
                sf = jnp.where(kv <= 0.0, s_neg, s_pos).astype(jnp.float32)
                s_v[b, pl.ds(g * 16, 16)] = sf
                racc[...] = racc[...] + sf
                return 0

            lax.fori_loop(0, C // 16, group, 0)
            base = wid * ew + k * C
            pltpu.async_copy(s_v.at[b], s_hbm.at[pl.ds(base, C)],
                             sem_out[b])

        prefetch(0, 0)

        def iter2(k2i, _):
            keven = 2 * k2i
            kodd = keven + 1

            @pl.when(kodd < KCH)
            def _():
                prefetch(kodd, 1)

            drain_in(0)
            compute(keven, 0)

            @pl.when(keven + 2 < KCH)
            def _():
                prefetch(keven + 2, 0)

            @pl.when(kodd < KCH)
            def _():
                drain_in(1)
                compute(kodd, 1)

            return 0

        lax.fori_loop(0, (KCH + 1) // 2, iter2, 0)
        pltpu.make_async_copy(s_v.at[0], s_hbm.at[pl.ds(0, C)],
                              sem_out[0]).wait()
        pltpu.make_async_copy(s_v.at[1], s_hbm.at[pl.ds(0, C)],
                              sem_out[1]).wait()
        pltpu.sync_copy(racc, cnt_hbm.at[wid])

    return main


def _sc_scatter(E, N, MD, with_deg):
    ew = E // _NW
    MH = MD // 2
    mesh = plsc.VectorSubcoreMesh(core_axis_name="c", subcore_axis_name="s")

    NR = ew // _SUB
    acc_t = jax.ShapeDtypeStruct((_NC, _NPAD, MH), jnp.float32)
    deg_t = jax.ShapeDtypeStruct((_NC, _NPAD), jnp.float32)
    scratch = [
        pltpu.VMEM((NR, _SUB), jnp.int32),
        pltpu.VMEM((_C, MH), jnp.float32),
        pltpu.VMEM((_SUB, MH), jnp.float32),
        pltpu.VMEM_SHARED((_NPAD, MH), jnp.float32),
        pltpu.SemaphoreType.DMA,
    ]
    if with_deg:
        scratch += [
            pltpu.VMEM((_SUB,), jnp.float32),
            pltpu.VMEM((_SL,), jnp.float32),
            pltpu.VMEM_SHARED((_NPAD,), jnp.float32),
        ]

    @functools.partial(
        pl.kernel, mesh=mesh,
        compiler_params=pltpu.CompilerParams(needs_layout_passes=False,
                                             use_tc_tiling_on_sc=False),
        out_type=(acc_t, deg_t) if with_deg else acc_t,
        scratch_types=scratch)
    def scat(a_hbm, dst2_hbm, *refs):
        if with_deg:
            (acc_hbm, deg_hbm, idx_all, a_v, zacc, acc_sh, sem,
             ones1, zdeg, deg_sh) = refs
        else:
            acc_hbm, idx_all, a_v, zacc, acc_sh, sem = refs
        cid = lax.axis_index("c")
        sid = lax.axis_index("s")
        wid = sid * _NC + cid

        _zero_vmem(zacc, _SUB * MH // 16)
        for z in range(_SL // _SUB):
            pltpu.sync_copy(zacc,
                            acc_sh.at[pl.ds(sid * _SL + z * _SUB, _SUB)])
        if with_deg:
            _zero_vmem(zdeg, _SL // 16)

            def fill_ones(t, _):
                ones1[pl.ds(t * 16, 16)] = jnp.full((16,), 1.0, jnp.float32)
                return 0

            lax.fori_loop(0, _SUB // 16, fill_ones, 0)
            pltpu.sync_copy(zdeg, deg_sh.at[pl.ds(sid * _SL, _SL)])
        pltpu.sync_copy(dst2_hbm.at[pl.ds(wid * NR, NR)], idx_all)
        plsc.subcore_barrier()

        def chunk1(k, _):
            base = wid * ew + k * _C
            pltpu.sync_copy(a_hbm.at[pl.ds(base, _C)], a_v)
            for j in range(_NSUB):
                r = k * _NSUB + j
                pltpu.async_copy(a_v.at[pl.ds(j * _SUB, _SUB)],
                                 acc_sh.at[idx_all.at[r]], sem, add=True)
                if with_deg:
                    pltpu.async_copy(ones1, deg_sh.at[idx_all.at[r]], sem,
                                     add=True)
            for j in range(_NSUB):
                pltpu.make_async_copy(a_v.at[pl.ds(j * _SUB, _SUB)],
                                      acc_sh.at[idx_all.at[0]],
                                      sem).wait()
                if with_deg:
                    pltpu.make_async_copy(ones1,
                                          deg_sh.at[idx_all.at[0]],
                                          sem).wait()
            return 0

        lax.fori_loop(0, ew // _C, chunk1, 0)
        plsc.subcore_barrier()
        pltpu.sync_copy(acc_sh.at[pl.ds(sid * _SL, _SL)],
                        acc_hbm.at[cid, pl.ds(sid * _SL, _SL)])
        if with_deg:
            pltpu.sync_copy(deg_sh.at[pl.ds(sid * _SL, _SL)],
                            deg_hbm.at[cid, pl.ds(sid * _SL, _SL)])

    return scat


def kernel(edge_feat, timestamp, dst_idx, memory, last_update,
           W0, b0, ln_g, ln_b, W1, b1, time_w, time_b):
    E, EF = edge_feat.shape
    N, MD = memory.shape
    f32 = jnp.float32

    dst_i32 = dst_idx.astype(jnp.int32)
    dst2 = dst_i32.reshape(E // _SUB, _SUB)
    w = (ln_g * W1[0]).astype(f32)

    w0b_t = W0[:, EF:].T
    P = pl.pallas_call(
        _p_body,
        out_shape=jax.ShapeDtypeStruct((N, MD), f32),
    )(memory, w0b_t)

    lu_g = _sc_lu_gather(E)(dst2, last_update)

    BE = 1280
    w0a_t = W0[:, :EF].T
    grid = (E // BE,)
    A = pl.pallas_call(
        _a_body,
        grid=grid,
        in_specs=[
            pl.BlockSpec((BE, EF), lambda i: (i, 0)),
            pl.BlockSpec((BE, 1), lambda i: (i, 0)),
            pl.BlockSpec((BE, 1), lambda i: (i, 0)),
            pl.BlockSpec((1, EF), lambda i: (0, 0)),
            pl.BlockSpec((1, EF), lambda i: (0, 0)),
            pl.BlockSpec((EF, MD), lambda i: (0, 0)),
            pl.BlockSpec((1, MD), lambda i: (0, 0)),
        ],
        out_specs=(pl.BlockSpec((BE, MD // 2), lambda i: (i, 0)),
                   pl.BlockSpec((BE, MD // 2), lambda i: (i, 0))),
        out_shape=(jax.ShapeDtypeStruct((E, MD // 2), f32),
                   jax.ShapeDtypeStruct((E, MD // 2), f32)),
    )(edge_feat, timestamp.reshape(E, 1), lu_g.reshape(E, 1),
      time_w.reshape(1, EF), time_b.reshape(1, EF), w0a_t,
      b0.reshape(1, MD))
    A_lo, A_hi = A

    wg = jnp.sum(w)
    c0 = jnp.sum(ln_b * W1[0]) + b1[0]
    kcv = jnp.float32(_THETA) - c0
    par = jnp.concatenate([jnp.stack([wg, kcv, kcv * kcv]),
                           jnp.zeros((13,))]).astype(f32)
    s_ij, counts = _sc_main(E, N, MD)(A_lo, A_hi, P, dst2, w, par)
    acc_lo, deg = _sc_scatter(E, N, MD, True)(A_lo, dst2)
    acc_hi = _sc_scatter(E, N, MD, False)(A_hi, dst2)

    BR = 2000
    new_memory, rem = pl.pallas_call(
        _fin_body,
        grid=(N // BR,),
        in_specs=[
            pl.BlockSpec((_NC, BR, MD // 2), lambda i: (0, i, 0)),
            pl.BlockSpec((_NC, BR, MD // 2), lambda i: (0, i, 0)),
            pl.BlockSpec((BR, _NC), lambda i: (i, 0)),
            pl.BlockSpec((BR, MD), lambda i: (i, 0)),
            pl.BlockSpec((BR, MD), lambda i: (i, 0)),
            pl.BlockSpec((_NW, 16), lambda i: (0, 0)),
        ],
        out_shape=(jax.ShapeDtypeStruct((N, MD), f32),
                   jax.ShapeDtypeStruct((1, 1), f32)),
        out_specs=(pl.BlockSpec((BR, MD), lambda i: (i, 0)),
                   pl.BlockSpec(memory_space=pltpu.SMEM)),
    )(acc_lo, acc_hi, deg.T, P, memory, counts)

    penalty = jnp.float32(0.0)
    remain_edge_batch = rem[0, 0]
    total_edge_batch = E
    return (new_memory, penalty, remain_edge_batch, total_edge_batch, s_ij)
